# baseline (speedup 1.0000x reference)
import jax
import jax.numpy as jnp
from jax.experimental import pallas as pl
from jax.experimental.pallas import tpu as pltpu

N = 4096
FEAT = 128
HID = 16
NH = 9
NC = 27
BLK = 1024
NSPLIT = 2
KCH = N // NSPLIT


def _body(x_ref, *refs):
    adj_refs = refs[:NSPLIT]
    (w_gc_ref, b_gc_ref, w_mlp_ref, b_mlp_ref, w_cls_ref, b_cls_ref,
     out_ref, sup_ref) = refs[NSPLIT:]
    b = pl.program_id(0)
    i = pl.program_id(1)

    @pl.when(b == 0)
    def _():
        sup_ref[i] = jnp.dot(x_ref[i], w_gc_ref[i],
                             preferred_element_type=jnp.float32)

    sup = sup_ref[i]
    h = b_gc_ref[i] + jnp.zeros((BLK, HID), jnp.float32)
    for j, a_ref in enumerate(adj_refs):
        adj = a_ref[0]
        adj = jnp.where(jnp.isnan(adj), 0.0, adj)
        h = h + jnp.dot(adj, sup[j * KCH:(j + 1) * KCH, :],
                        preferred_element_type=jnp.float32)
    t = jnp.tanh(jnp.dot(h, w_mlp_ref[i], preferred_element_type=jnp.float32)
                 + b_mlp_ref[i])
    w_cls_i = w_cls_ref[pl.ds(i * NH, NH), :]
    contrib = jnp.dot(t, w_cls_i, preferred_element_type=jnp.float32)

    @pl.when(i == 0)
    def _():
        out_ref[...] = contrib + b_cls_ref[0]

    @pl.when(i != 0)
    def _():
        out_ref[...] += contrib


@jax.jit
def kernel(x, adjs, W_gc, b_gc, W_mlp, b_mlp, W_cls, b_cls):
    nb = N // BLK
    out = pl.pallas_call(
        _body,
        grid=(nb, 3),
        in_specs=[
            pl.BlockSpec((3, N, FEAT), lambda b, i: (0, 0, 0)),
        ] + [
            pl.BlockSpec((1, BLK, KCH),
                         (lambda jj: lambda b, i: (i, b, jj))(j))
            for j in range(NSPLIT)
        ] + [
            pl.BlockSpec((3, FEAT, HID), lambda b, i: (0, 0, 0)),
            pl.BlockSpec((3, HID), lambda b, i: (0, 0)),
            pl.BlockSpec((3, HID, NH), lambda b, i: (0, 0, 0)),
            pl.BlockSpec((3, NH), lambda b, i: (0, 0)),
            pl.BlockSpec((3 * NH, NC), lambda b, i: (0, 0)),
            pl.BlockSpec((1, NC), lambda b, i: (0, 0)),
        ],
        out_specs=pl.BlockSpec((BLK, NC), lambda b, i: (b, 0)),
        out_shape=jax.ShapeDtypeStruct((N, NC), jnp.float32),
        scratch_shapes=[pltpu.VMEM((3, N, HID), jnp.float32)],
    )(x, *([adjs] * NSPLIT), W_gc, b_gc, W_mlp, b_mlp, W_cls,
      b_cls.reshape(1, NC))
    return out


# speedup vs baseline: 1.0035x; 1.0035x over previous
import jax
import jax.numpy as jnp
from jax.experimental import pallas as pl
from jax.experimental.pallas import tpu as pltpu

N = 4096
FEAT = 128
HID = 16
NH = 9
NC = 27
BLK = 1024
NSPLIT = 2
RCH = BLK // NSPLIT


def _body(x_ref, *refs):
    adj_refs = refs[:NSPLIT]
    (w_gc_ref, b_gc_ref, w_mlp_ref, b_mlp_ref, w_cls_ref, b_cls_ref,
     out_ref, sup_ref) = refs[NSPLIT:]
    b = pl.program_id(0)
    i = pl.program_id(1)

    @pl.when(b == 0)
    def _():
        sup_ref[i] = jnp.dot(x_ref[i], w_gc_ref[i],
                             preferred_element_type=jnp.float32)

    sup = sup_ref[i]
    hs = []
    for a_ref in adj_refs:
        adj = a_ref[0]
        adj = jnp.where(jnp.isnan(adj), 0.0, adj)
        hs.append(jnp.dot(adj, sup, preferred_element_type=jnp.float32))
    h = jnp.concatenate(hs, axis=0) + b_gc_ref[i]
    t = jnp.tanh(jnp.dot(h, w_mlp_ref[i], preferred_element_type=jnp.float32)
                 + b_mlp_ref[i])
    w_cls_i = w_cls_ref[pl.ds(i * NH, NH), :]
    contrib = jnp.dot(t, w_cls_i, preferred_element_type=jnp.float32)

    @pl.when(i == 0)
    def _():
        out_ref[...] = contrib + b_cls_ref[0]

    @pl.when(i != 0)
    def _():
        out_ref[...] += contrib


@jax.jit
def kernel(x, adjs, W_gc, b_gc, W_mlp, b_mlp, W_cls, b_cls):
    nb = N // BLK
    out = pl.pallas_call(
        _body,
        grid=(nb, 3),
        in_specs=[
            pl.BlockSpec((3, N, FEAT), lambda b, i: (0, 0, 0)),
        ] + [
            pl.BlockSpec((1, RCH, N),
                         (lambda jj: lambda b, i: (i, NSPLIT * b + jj, 0))(j))
            for j in range(NSPLIT)
        ] + [
            pl.BlockSpec((3, FEAT, HID), lambda b, i: (0, 0, 0)),
            pl.BlockSpec((3, HID), lambda b, i: (0, 0)),
            pl.BlockSpec((3, HID, NH), lambda b, i: (0, 0, 0)),
            pl.BlockSpec((3, NH), lambda b, i: (0, 0)),
            pl.BlockSpec((3 * NH, NC), lambda b, i: (0, 0)),
            pl.BlockSpec((1, NC), lambda b, i: (0, 0)),
        ],
        out_specs=pl.BlockSpec((BLK, NC), lambda b, i: (b, 0)),
        out_shape=jax.ShapeDtypeStruct((N, NC), jnp.float32),
        scratch_shapes=[pltpu.VMEM((3, N, HID), jnp.float32)],
    )(x, *([adjs] * NSPLIT), W_gc, b_gc, W_mlp, b_mlp, W_cls,
      b_cls.reshape(1, NC))
    return out


# speedup vs baseline: 1.0097x; 1.0061x over previous
import jax
import jax.numpy as jnp
from jax.experimental import pallas as pl
from jax.experimental.pallas import tpu as pltpu

N = 4096
FEAT = 128
HID = 16
NH = 9
NC = 27
BLK = 1024


def _body(x_ref, adj_ref, w_gc_ref, b_gc_ref, w_mlp_ref, b_mlp_ref,
          w_cls_ref, b_cls_ref, out_ref, sup_ref):
    b = pl.program_id(0)
    i = pl.program_id(1)

    @pl.when(b == 0)
    def _():
        sup_ref[i] = jnp.dot(x_ref[i], w_gc_ref[i],
                             preferred_element_type=jnp.float32)

    adj = adj_ref[0]
    adj = jnp.where(jnp.isnan(adj), 0.0, adj)
    h = jnp.dot(adj, sup_ref[i], preferred_element_type=jnp.float32)
    h = h + b_gc_ref[i]
    t = jnp.tanh(jnp.dot(h, w_mlp_ref[i], preferred_element_type=jnp.float32)
                 + b_mlp_ref[i])
    w_cls_i = w_cls_ref[pl.ds(i * NH, NH), :]
    contrib = jnp.dot(t, w_cls_i, preferred_element_type=jnp.float32)

    @pl.when(i == 0)
    def _():
        out_ref[...] = contrib + b_cls_ref[0]

    @pl.when(i != 0)
    def _():
        out_ref[...] += contrib


@jax.jit
def kernel(x, adjs, W_gc, b_gc, W_mlp, b_mlp, W_cls, b_cls):
    nb = N // BLK
    out = pl.pallas_call(
        _body,
        grid=(nb, 3),
        in_specs=[
            pl.BlockSpec((3, N, FEAT), lambda b, i: (0, 0, 0)),
            pl.BlockSpec((1, BLK, N), lambda b, i: (i, b, 0)),
            pl.BlockSpec((3, FEAT, HID), lambda b, i: (0, 0, 0)),
            pl.BlockSpec((3, HID), lambda b, i: (0, 0)),
            pl.BlockSpec((3, HID, NH), lambda b, i: (0, 0, 0)),
            pl.BlockSpec((3, NH), lambda b, i: (0, 0)),
            pl.BlockSpec((3 * NH, NC), lambda b, i: (0, 0)),
            pl.BlockSpec((1, NC), lambda b, i: (0, 0)),
        ],
        out_specs=pl.BlockSpec((BLK, NC), lambda b, i: (b, 0)),
        out_shape=jax.ShapeDtypeStruct((N, NC), jnp.float32),
        scratch_shapes=[pltpu.VMEM((3, N, HID), jnp.float32)],
    )(x, adjs, W_gc, b_gc, W_mlp, b_mlp, W_cls, b_cls.reshape(1, NC))
    return out
